# baseline (device time: 86839 ns/iter reference)
import jax
import jax.numpy as jnp
from jax import lax
from jax.experimental import pallas as pl
from jax.experimental.pallas import tpu as pltpu

N_DEV = 16
B, SQ, D = 4, 256, 1024
HQ_LOC, DH = 8, 128
KV_LOC = 2
ROWS = B * SQ
CHUNK = ROWS // N_DEV
SCALE = 0.08838834764831843

_MESH = pl.DeviceIdType.MESH


def kernel(x, Wq, Wo, Wk, Wv):
    def body(x_ref, wq_ref, wo_ref, wk_hbm, wv_hbm, out_ref,
             wk_s, wv_s, q_ref, k_ref, v_ref, rs_send, rs_stage,
             ag_send, ag_stage,
             local_sems, rs_ssem, rs_rsem, ag_ssem, ag_rsem):
        d = lax.axis_index("i")

        cp_k = pltpu.make_async_copy(
            wk_hbm.at[:, pl.ds(d * KV_LOC * DH, KV_LOC * DH)],
            wk_s, local_sems.at[0])
        cp_v = pltpu.make_async_copy(
            wv_hbm.at[:, pl.ds(d * KV_LOC * DH, KV_LOC * DH)],
            wv_s, local_sems.at[1])
        cp_k.start()
        cp_v.start()

        barrier = pltpu.get_barrier_semaphore()
        for k in range(1, N_DEV):
            pl.semaphore_signal(barrier, inc=1, device_id=((d + k) % N_DEV,),
                                device_id_type=_MESH)
        pl.semaphore_wait(barrier, N_DEV - 1)

        xb = x_ref[...].astype(jnp.bfloat16)
        wo_b = wo_ref[...].astype(jnp.bfloat16)
        q_ref[...] = (jnp.dot(xb, wq_ref[...].astype(jnp.bfloat16),
                              preferred_element_type=jnp.float32)
                      * SCALE).astype(jnp.bfloat16)
        cp_k.wait()
        cp_v.wait()
        k_ref[...] = jnp.dot(xb, wk_s[...].astype(jnp.bfloat16),
                             preferred_element_type=jnp.float32
                             ).astype(jnp.bfloat16)
        v_ref[...] = jnp.dot(xb, wv_s[...].astype(jnp.bfloat16),
                             preferred_element_type=jnp.float32
                             ).astype(jnp.bfloat16)

        def chunk_partial(c):
            q_c = q_ref[pl.ds(c * CHUNK, CHUNK), :]
            b_t = c // (SQ // CHUNK)
            k_b = k_ref[pl.ds(b_t * SQ, SQ), :]
            v_b = v_ref[pl.ds(b_t * SQ, SQ), :]
            outs = []
            for h in range(HQ_LOC):
                g = h // 4
                q = q_c[:, h * DH:(h + 1) * DH]
                kk = k_b[:, g * DH:(g + 1) * DH]
                vv = v_b[:, g * DH:(g + 1) * DH]
                s = lax.dot_general(q, kk, (((1,), (1,)), ((), ())),
                                    preferred_element_type=jnp.float32)
                m = jnp.max(s, axis=1, keepdims=True)
                p = jnp.exp(s - m)
                den = jnp.sum(p, axis=1, keepdims=True)
                pn = (p / den).astype(jnp.bfloat16)
                outs.append(jnp.dot(pn, vv,
                                    preferred_element_type=jnp.float32))
            attn_c = jnp.concatenate(outs, axis=1).astype(jnp.bfloat16)
            return jnp.dot(attn_c, wo_b, preferred_element_type=jnp.float32)

        rs_desc = []
        for j in range(N_DEV - 1):
            c_t = (d + 1 + j) % N_DEV
            pc = chunk_partial(c_t)
            rs_send[pl.ds(c_t * CHUNK, CHUNK), :] = pc.astype(jnp.bfloat16)
            r = pltpu.make_async_remote_copy(
                src_ref=rs_send.at[pl.ds(c_t * CHUNK, CHUNK), :],
                dst_ref=rs_stage.at[d],
                send_sem=rs_ssem, recv_sem=rs_rsem,
                device_id=(c_t,), device_id_type=_MESH)
            r.start()
            rs_desc.append(r)
        own = chunk_partial(d)

        for r in rs_desc:
            r.wait_recv()

        rs_stage[d] = jnp.zeros((CHUNK, D), jnp.bfloat16)
        summed = own + jnp.sum(rs_stage[...].astype(jnp.float32), axis=0)
        out_ref[pl.ds(d * CHUNK, CHUNK), :] = summed
        ag_send[...] = summed.astype(jnp.bfloat16)

        ag_desc = []
        for k in range(1, N_DEV):
            dest = (d + k) % N_DEV
            r = pltpu.make_async_remote_copy(
                src_ref=ag_send, dst_ref=ag_stage.at[d],
                send_sem=ag_ssem, recv_sem=ag_rsem,
                device_id=(dest,), device_id_type=_MESH)
            r.start()
            ag_desc.append(r)
        for r in ag_desc:
            r.wait_recv()

        for k in range(N_DEV):
            @pl.when(k != d)
            def _(k=k):
                out_ref[k * CHUNK:(k + 1) * CHUNK, :] = \
                    ag_stage[k].astype(jnp.float32)

        for r in rs_desc:
            r.wait_send()
        for r in ag_desc:
            r.wait_send()

    out = pl.pallas_call(
        body,
        out_shape=jax.ShapeDtypeStruct((ROWS, D), jnp.float32),
        in_specs=[
            pl.BlockSpec(memory_space=pltpu.VMEM),
            pl.BlockSpec(memory_space=pltpu.VMEM),
            pl.BlockSpec(memory_space=pltpu.VMEM),
            pl.BlockSpec(memory_space=pltpu.HBM),
            pl.BlockSpec(memory_space=pltpu.HBM),
        ],
        out_specs=pl.BlockSpec(memory_space=pltpu.VMEM),
        scratch_shapes=[
            pltpu.VMEM((D, KV_LOC * DH), jnp.float32),
            pltpu.VMEM((D, KV_LOC * DH), jnp.float32),
            pltpu.VMEM((ROWS, HQ_LOC * DH), jnp.bfloat16),
            pltpu.VMEM((ROWS, KV_LOC * DH), jnp.bfloat16),
            pltpu.VMEM((ROWS, KV_LOC * DH), jnp.bfloat16),
            pltpu.VMEM((ROWS, D), jnp.bfloat16),
            pltpu.VMEM((N_DEV, CHUNK, D), jnp.bfloat16),
            pltpu.VMEM((CHUNK, D), jnp.bfloat16),
            pltpu.VMEM((N_DEV, CHUNK, D), jnp.bfloat16),
            pltpu.SemaphoreType.DMA((2,)),
            pltpu.SemaphoreType.DMA,
            pltpu.SemaphoreType.DMA,
            pltpu.SemaphoreType.DMA,
            pltpu.SemaphoreType.DMA,
        ],
        compiler_params=pltpu.CompilerParams(collective_id=0),
    )(x.reshape(ROWS, D), Wq, Wo, Wk, Wv)
    return out.reshape(B, SQ, D)


# device time: 63159 ns/iter; 1.3749x vs baseline; 1.3749x over previous
import jax
import jax.numpy as jnp
from jax import lax
from jax.experimental import pallas as pl
from jax.experimental.pallas import tpu as pltpu

N_DEV = 16
B, SQ, D = 4, 256, 1024
HQ_LOC, DH = 8, 128
KV_LOC = 2
ROWS = B * SQ
PIECE = SQ // N_DEV
SCALE = 0.08838834764831843

_MESH = pl.DeviceIdType.MESH


def kernel(x, Wq, Wo, Wk, Wv):
    def body(x_ref, wq_ref, wo_ref, wk_hbm, wv_hbm, out_ref,
             wk_s, wv_s, rs_send, rs_stage,
             local_sems, rs_ssem, rs_rsems, ag_ssem, ag_rsem):
        d = lax.axis_index("i")

        cp_k = pltpu.make_async_copy(
            wk_hbm.at[:, pl.ds(d * KV_LOC * DH, KV_LOC * DH)],
            wk_s, local_sems.at[0])
        cp_v = pltpu.make_async_copy(
            wv_hbm.at[:, pl.ds(d * KV_LOC * DH, KV_LOC * DH)],
            wv_s, local_sems.at[1])
        cp_k.start()
        cp_v.start()

        rs_stage[pl.ds(d * B, B)] = jnp.zeros((B, PIECE, D), jnp.bfloat16)

        barrier = pltpu.get_barrier_semaphore()
        for k in range(1, N_DEV):
            pl.semaphore_signal(barrier, inc=1, device_id=((d + k) % N_DEV,),
                                device_id_type=_MESH)
        pl.semaphore_wait(barrier, N_DEV - 1)

        wq_b = wq_ref[...].astype(jnp.bfloat16)
        wo_b = wo_ref[...].astype(jnp.bfloat16)
        cp_k.wait()
        cp_v.wait()
        wk_b = wk_s[...].astype(jnp.bfloat16)
        wv_b = wv_s[...].astype(jnp.bfloat16)

        rs_desc, ag_desc = [], []

        def piece_wait_recv(sem):
            dummy = rs_send.at[pl.ds(0, PIECE), :]
            pltpu.make_async_remote_copy(
                src_ref=dummy, dst_ref=dummy, send_sem=rs_ssem,
                recv_sem=sem, device_id=(d,), device_id_type=_MESH,
            ).wait_recv()

        def finalize(b):
            for _ in range(N_DEV - 1):
                piece_wait_recv(rs_rsems.at[b])
            own = rs_send[pl.ds(b * SQ + d * PIECE, PIECE), :]
            acc = own.astype(jnp.float32)
            for k in range(N_DEV):
                acc = acc + rs_stage[k * B + b].astype(jnp.float32)
            out_ref[pl.ds(b * SQ + d * PIECE, PIECE), :] = \
                acc.astype(jnp.bfloat16)
            src = out_ref.at[pl.ds(b * SQ + d * PIECE, PIECE), :]
            for k in range(1, N_DEV):
                r = pltpu.make_async_remote_copy(
                    src_ref=src, dst_ref=src,
                    send_sem=ag_ssem, recv_sem=ag_rsem,
                    device_id=((d + k) % N_DEV,), device_id_type=_MESH)
                r.start()
                ag_desc.append(r)

        for b in range(B):
            r0 = b * SQ
            xb = x_ref[pl.ds(r0, SQ), :].astype(jnp.bfloat16)
            q_b = (jnp.dot(xb, wq_b, preferred_element_type=jnp.float32)
                   * SCALE).astype(jnp.bfloat16)
            k_b = jnp.dot(xb, wk_b,
                          preferred_element_type=jnp.float32
                          ).astype(jnp.bfloat16)
            v_b = jnp.dot(xb, wv_b,
                          preferred_element_type=jnp.float32
                          ).astype(jnp.bfloat16)
            outs = []
            for h in range(HQ_LOC):
                g = h // 4
                q = q_b[:, h * DH:(h + 1) * DH]
                kk = k_b[:, g * DH:(g + 1) * DH]
                vv = v_b[:, g * DH:(g + 1) * DH]
                s = lax.dot_general(q, kk, (((1,), (1,)), ((), ())),
                                    preferred_element_type=jnp.float32)
                m = jnp.max(s, axis=1, keepdims=True)
                p = jnp.exp(s - m)
                den = jnp.sum(p, axis=1, keepdims=True)
                pn = (p / den).astype(jnp.bfloat16)
                outs.append(jnp.dot(pn, vv,
                                    preferred_element_type=jnp.float32))
            attn_b = jnp.concatenate(outs, axis=1).astype(jnp.bfloat16)
            pc = jnp.dot(attn_b, wo_b, preferred_element_type=jnp.float32)
            rs_send[pl.ds(r0, SQ), :] = pc.astype(jnp.bfloat16)

            for k in range(1, N_DEV):
                dest = (d + k) % N_DEV
                r = pltpu.make_async_remote_copy(
                    src_ref=rs_send.at[pl.ds(r0 + dest * PIECE, PIECE), :],
                    dst_ref=rs_stage.at[d * B + b],
                    send_sem=rs_ssem, recv_sem=rs_rsems.at[b],
                    device_id=(dest,), device_id_type=_MESH)
                r.start()
                rs_desc.append(r)

            if b >= 1:
                finalize(b - 1)
        finalize(B - 1)

        for _ in range(B * (N_DEV - 1)):
            piece_wait_recv(ag_rsem)

        for r in rs_desc:
            r.wait_send()
        for r in ag_desc:
            r.wait_send()

    out = pl.pallas_call(
        body,
        out_shape=jax.ShapeDtypeStruct((ROWS, D), jnp.bfloat16),
        in_specs=[
            pl.BlockSpec(memory_space=pltpu.VMEM),
            pl.BlockSpec(memory_space=pltpu.VMEM),
            pl.BlockSpec(memory_space=pltpu.VMEM),
            pl.BlockSpec(memory_space=pltpu.HBM),
            pl.BlockSpec(memory_space=pltpu.HBM),
        ],
        out_specs=pl.BlockSpec(memory_space=pltpu.VMEM),
        scratch_shapes=[
            pltpu.VMEM((D, KV_LOC * DH), jnp.float32),
            pltpu.VMEM((D, KV_LOC * DH), jnp.float32),
            pltpu.VMEM((ROWS, D), jnp.bfloat16),
            pltpu.VMEM((N_DEV * B, PIECE, D), jnp.bfloat16),
            pltpu.SemaphoreType.DMA((2,)),
            pltpu.SemaphoreType.DMA,
            pltpu.SemaphoreType.DMA((B,)),
            pltpu.SemaphoreType.DMA,
            pltpu.SemaphoreType.DMA,
        ],
        compiler_params=pltpu.CompilerParams(collective_id=0),
    )(x.reshape(ROWS, D), Wq, Wo, Wk, Wv)
    return out.astype(jnp.float32).reshape(B, SQ, D)


# device time: 62088 ns/iter; 1.3986x vs baseline; 1.0172x over previous
import jax
import jax.numpy as jnp
from jax import lax
from jax.experimental import pallas as pl
from jax.experimental.pallas import tpu as pltpu

N_DEV = 16
B, SQ, D = 4, 256, 1024
HQ_LOC, DH = 8, 128
KV_LOC = 2
ROWS = B * SQ
PIECE = SQ // N_DEV
SCALE = 0.08838834764831843

_MESH = pl.DeviceIdType.MESH


def kernel(x, Wq, Wo, Wk, Wv):
    def body(x_ref, wq_ref, wo_ref, wk_hbm, wv_hbm, out_ref,
             wk_s, wv_s, rs_send, rs_stage,
             local_sems, rs_ssem, rs_rsems, ag_ssem, ag_rsem):
        d = lax.axis_index("i")

        cp_k = pltpu.make_async_copy(
            wk_hbm.at[:, pl.ds(d * KV_LOC * DH, KV_LOC * DH)],
            wk_s, local_sems.at[0])
        cp_v = pltpu.make_async_copy(
            wv_hbm.at[:, pl.ds(d * KV_LOC * DH, KV_LOC * DH)],
            wv_s, local_sems.at[1])
        cp_k.start()
        cp_v.start()

        rs_stage[pl.ds(d * B, B)] = jnp.zeros((B, PIECE, D), jnp.bfloat16)

        barrier = pltpu.get_barrier_semaphore()
        for k in range(1, N_DEV):
            pl.semaphore_signal(barrier, inc=1, device_id=((d + k) % N_DEV,),
                                device_id_type=_MESH)
        pl.semaphore_wait(barrier, N_DEV - 1)

        wq_b = wq_ref[...].astype(jnp.bfloat16)
        wo_b = wo_ref[...].astype(jnp.bfloat16)
        cp_k.wait()
        cp_v.wait()
        wk_b = wk_s[...].astype(jnp.bfloat16)
        wv_b = wv_s[...].astype(jnp.bfloat16)

        rs_desc, ag_desc = [], []

        def piece_wait_recv(sem):
            dummy = rs_send.at[pl.ds(0, PIECE), :]
            pltpu.make_async_remote_copy(
                src_ref=dummy, dst_ref=dummy, send_sem=rs_ssem,
                recv_sem=sem, device_id=(d,), device_id_type=_MESH,
            ).wait_recv()

        def finalize(b):
            for _ in range(N_DEV - 1):
                piece_wait_recv(rs_rsems.at[b])
            own = rs_send[pl.ds(b * SQ + d * PIECE, PIECE), :]
            acc = own.astype(jnp.float32)
            for k in range(N_DEV):
                acc = acc + rs_stage[k * B + b].astype(jnp.float32)
            out_ref[pl.ds(b * SQ + d * PIECE, PIECE), :] = \
                acc.astype(jnp.bfloat16)
            src = out_ref.at[pl.ds(b * SQ + d * PIECE, PIECE), :]
            for k in range(1, N_DEV):
                r = pltpu.make_async_remote_copy(
                    src_ref=src, dst_ref=src,
                    send_sem=ag_ssem, recv_sem=ag_rsem,
                    device_id=((d + k) % N_DEV,), device_id_type=_MESH)
                r.start()
                ag_desc.append(r)

        for b in range(B):
            r0 = b * SQ
            xb = x_ref[pl.ds(r0, SQ), :].astype(jnp.bfloat16)
            q_b = (jnp.dot(xb, wq_b, preferred_element_type=jnp.float32)
                   * SCALE).astype(jnp.bfloat16)
            k_b = jnp.dot(xb, wk_b,
                          preferred_element_type=jnp.float32
                          ).astype(jnp.bfloat16)
            v_b = jnp.dot(xb, wv_b,
                          preferred_element_type=jnp.float32
                          ).astype(jnp.bfloat16)
            outs = []
            for h in range(HQ_LOC):
                g = h // 4
                q = q_b[:, h * DH:(h + 1) * DH]
                kk = k_b[:, g * DH:(g + 1) * DH]
                vv = v_b[:, g * DH:(g + 1) * DH]
                s = lax.dot_general(q, kk, (((1,), (1,)), ((), ())),
                                    preferred_element_type=jnp.float32)
                m = jnp.max(s, axis=1, keepdims=True)
                p = jnp.exp(s - m)
                den = jnp.sum(p, axis=1, keepdims=True)
                pn = (p / den).astype(jnp.bfloat16)
                outs.append(jnp.dot(pn, vv,
                                    preferred_element_type=jnp.float32))
            attn_b = jnp.concatenate(outs, axis=1).astype(jnp.bfloat16)
            pc = jnp.dot(attn_b, wo_b, preferred_element_type=jnp.float32)
            rs_send[pl.ds(r0, SQ), :] = pc.astype(jnp.bfloat16)

            for k in range(1, N_DEV):
                dest = (d + k) % N_DEV
                r = pltpu.make_async_remote_copy(
                    src_ref=rs_send.at[pl.ds(r0 + dest * PIECE, PIECE), :],
                    dst_ref=rs_stage.at[d * B + b],
                    send_sem=rs_ssem, recv_sem=rs_rsems.at[b],
                    device_id=(dest,), device_id_type=_MESH)
                r.start()
                rs_desc.append(r)

            if b >= 1:
                finalize(b - 1)
        finalize(B - 1)

        for _ in range(B * (N_DEV - 1)):
            piece_wait_recv(ag_rsem)

        for r in rs_desc:
            r.wait_send()
        for r in ag_desc:
            r.wait_send()

    out = pl.pallas_call(
        body,
        out_shape=jax.ShapeDtypeStruct((ROWS, D), jnp.bfloat16),
        in_specs=[
            pl.BlockSpec(memory_space=pltpu.VMEM),
            pl.BlockSpec(memory_space=pltpu.VMEM),
            pl.BlockSpec(memory_space=pltpu.VMEM),
            pl.BlockSpec(memory_space=pltpu.HBM),
            pl.BlockSpec(memory_space=pltpu.HBM),
        ],
        out_specs=pl.BlockSpec(memory_space=pltpu.VMEM),
        scratch_shapes=[
            pltpu.VMEM((D, KV_LOC * DH), jnp.float32),
            pltpu.VMEM((D, KV_LOC * DH), jnp.float32),
            pltpu.VMEM((ROWS, D), jnp.bfloat16),
            pltpu.VMEM((N_DEV * B, PIECE, D), jnp.bfloat16),
            pltpu.SemaphoreType.DMA((2,)),
            pltpu.SemaphoreType.DMA,
            pltpu.SemaphoreType.DMA((B,)),
            pltpu.SemaphoreType.DMA,
            pltpu.SemaphoreType.DMA,
        ],
        compiler_params=pltpu.CompilerParams(collective_id=0),
    )(x.reshape(ROWS, D), Wq, Wo, Wk, Wv)
    return out.reshape(B, SQ, D)
